# baseline (speedup 1.0000x reference)
import jax
import jax.numpy as jnp
from jax.experimental import pallas as pl
from jax.experimental.pallas import tpu as pltpu


def _routed_dense_body(combine_ref, x_ref, w13_ref, w2_ref, out_ref):
    e = pl.program_id(0)
    t = pl.program_id(1)
    tile_t = x_ref.shape[0]
    gate_up = jax.lax.dot_general(
        x_ref[...], w13_ref[0],
        dimension_numbers=(((1,), (1,)), ((), ())),
        preferred_element_type=jnp.float32)
    gate, up = jnp.split(gate_up, 2, axis=-1)
    act = (jax.nn.silu(gate) * up).astype(x_ref.dtype)
    down = jax.lax.dot_general(
        act, w2_ref[0],
        dimension_numbers=(((1,), (1,)), ((), ())),
        preferred_element_type=jnp.float32)
    contrib = down * combine_ref[0]

    @pl.when(e == 0)
    def _():
        out_ref[pl.ds(t * tile_t, tile_t), :] = contrib

    @pl.when(e != 0)
    def _():
        out_ref[pl.ds(t * tile_t, tile_t), :] += contrib


def _shared_mlp_body(x_ref, wgu_ref, wd_ref, out_ref):
    gate_up = jax.lax.dot_general(
        x_ref[...], wgu_ref[...],
        dimension_numbers=(((1,), (1,)), ((), ())),
        preferred_element_type=jnp.float32)
    gate, up = jnp.split(gate_up, 2, axis=-1)
    act = (jax.nn.silu(gate) * up).astype(x_ref.dtype)
    out_ref[...] = jax.lax.dot_general(
        act, wd_ref[...],
        dimension_numbers=(((1,), (1,)), ((), ())),
        preferred_element_type=jnp.float32)


def kernel(hidden_states, topk_idx, topk_weights, w13, w2,
           shared_gate_up, shared_down):
    T, H = hidden_states.shape
    E, I2, _ = w13.shape
    I = I2 // 2
    tile_t = min(256, T)
    nt = T // tile_t

    x_bf = hidden_states.astype(jnp.bfloat16)
    w13_bf = w13.astype(jnp.bfloat16)
    w2_bf = w2.astype(jnp.bfloat16)
    sgu_bf = shared_gate_up.astype(jnp.bfloat16)
    sd_bf = shared_down.astype(jnp.bfloat16)

    combine = jnp.zeros((T, E), jnp.float32).at[
        jnp.arange(T)[:, None], topk_idx.astype(jnp.int32)
    ].add(topk_weights)
    combine_etc = combine.T.reshape(E, T, 1)

    combined_x = pl.pallas_call(
        _routed_dense_body,
        grid=(E, nt),
        in_specs=[
            pl.BlockSpec((1, tile_t, 1), lambda e, t: (e, t, 0)),
            pl.BlockSpec((tile_t, H), lambda e, t: (t, 0)),
            pl.BlockSpec((1, I2, H), lambda e, t: (e, 0, 0)),
            pl.BlockSpec((1, H, I), lambda e, t: (e, 0, 0)),
        ],
        out_specs=pl.BlockSpec((T, H), lambda e, t: (0, 0)),
        out_shape=jax.ShapeDtypeStruct((T, H), jnp.float32),
        compiler_params=pltpu.CompilerParams(
            dimension_semantics=("arbitrary", "arbitrary")),
    )(combine_etc, x_bf, w13_bf, w2_bf)

    shared_output = pl.pallas_call(
        _shared_mlp_body,
        grid=(nt,),
        in_specs=[
            pl.BlockSpec((tile_t, H), lambda t: (t, 0)),
            pl.BlockSpec((I2, H), lambda t: (0, 0)),
            pl.BlockSpec((H, I), lambda t: (0, 0)),
        ],
        out_specs=pl.BlockSpec((tile_t, H), lambda t: (t, 0)),
        out_shape=jax.ShapeDtypeStruct((T, H), jnp.float32),
    )(x_bf, sgu_bf, sd_bf)

    return (combined_x, shared_output)


# speedup vs baseline: 1.0023x; 1.0023x over previous
import functools

import jax
import jax.numpy as jnp
from jax import lax
from jax.experimental import pallas as pl
from jax.experimental.pallas import tpu as pltpu
from jax.experimental.pallas import tpu_sc as plsc

_NC = 2
_NS = 16
_NW = _NC * _NS
_L = 16


def _dispatch_sc(x, gtok, pad_a):
    t, h = x.shape
    b_per_w = pad_a // _NW
    ch = min(64, b_per_w)
    mesh = plsc.VectorSubcoreMesh(core_axis_name="c", subcore_axis_name="s")

    @functools.partial(
        pl.kernel, mesh=mesh,
        out_type=jax.ShapeDtypeStruct((pad_a, h), jnp.float32),
        scratch_types=[
            pltpu.VMEM((b_per_w,), jnp.int32),
            pltpu.VMEM((ch, h), jnp.float32),
            pltpu.SemaphoreType.DMA,
        ],
    )
    def k(x_hbm, idx_hbm, out_hbm, idx_v, rows_v, sem):
        wid = lax.axis_index("s") * _NC + lax.axis_index("c")
        base = wid * b_per_w
        pltpu.sync_copy(idx_hbm.at[pl.ds(base, b_per_w)], idx_v)

        @pl.loop(0, b_per_w // ch)
        def _(c):
            pltpu.async_copy(
                x_hbm.at[idx_v.at[pl.ds(c * ch, ch)]], rows_v, sem).wait()
            pltpu.sync_copy(rows_v, out_hbm.at[pl.ds(base + c * ch, ch)])

    return k(x, gtok)


def _combine_sc(dg, pos0, pos1):
    t = pos0.shape[0]
    h = dg.shape[1]
    t_per_w = t // _NW
    cht = min(32, t_per_w)
    mesh = plsc.VectorSubcoreMesh(core_axis_name="c", subcore_axis_name="s")

    @functools.partial(
        pl.kernel, mesh=mesh,
        out_type=jax.ShapeDtypeStruct((t, h), jnp.float32),
        scratch_types=[
            pltpu.VMEM((t_per_w,), jnp.int32),
            pltpu.VMEM((t_per_w,), jnp.int32),
            pltpu.VMEM((cht, h), jnp.float32),
            pltpu.VMEM((cht, h), jnp.float32),
            pltpu.SemaphoreType.DMA,
        ],
    )
    def k(dg_hbm, p0_hbm, p1_hbm, out_hbm, p0_v, p1_v, buf0, buf1, sem):
        wid = lax.axis_index("s") * _NC + lax.axis_index("c")
        base = wid * t_per_w
        pltpu.sync_copy(p0_hbm.at[pl.ds(base, t_per_w)], p0_v)
        pltpu.sync_copy(p1_hbm.at[pl.ds(base, t_per_w)], p1_v)

        @pl.loop(0, t_per_w // cht)
        def _(c):
            cp0 = pltpu.async_copy(
                dg_hbm.at[p0_v.at[pl.ds(c * cht, cht)]], buf0, sem)
            cp1 = pltpu.async_copy(
                dg_hbm.at[p1_v.at[pl.ds(c * cht, cht)]], buf1, sem)
            cp0.wait()
            cp1.wait()

            @pl.loop(0, cht)
            def _(r):
                @pl.loop(0, h // (4 * _L))
                def _(q):
                    for u in range(4):
                        slc = (pl.ds(r, 1), pl.ds(q * 4 * _L + u * _L, _L))
                        buf0.at[*slc][...] = (
                            buf0.at[*slc][...] + buf1.at[*slc][...])

            pltpu.sync_copy(buf0, out_hbm.at[pl.ds(base + c * cht, cht)])

    return k(dg, pos0, pos1)


def _grouped_gemm_body(te_ref, nt_ref, xg_ref, w13_ref, w2_ref, ws_ref,
                       dg_ref):
    i = pl.program_id(0)

    @pl.when(i < nt_ref[0])
    def _():
        x = xg_ref[...].astype(jnp.bfloat16)
        gate_up = jax.lax.dot_general(
            x, w13_ref[0],
            dimension_numbers=(((1,), (1,)), ((), ())),
            preferred_element_type=jnp.float32)
        gate, up = jnp.split(gate_up, 2, axis=-1)
        act = (jax.nn.silu(gate) * up).astype(jnp.bfloat16)
        down = jax.lax.dot_general(
            act, w2_ref[0],
            dimension_numbers=(((1,), (1,)), ((), ())),
            preferred_element_type=jnp.float32)
        dg_ref[...] = down * ws_ref[0]


def _grouped_gemm(xg, w13_bf, w2_bf, wslot, tile_expert, nt_valid,
                  tile, maxtiles):
    pad_a, h = xg.shape
    e, i2, _ = w13_bf.shape
    i_dim = i2 // 2
    ws3 = wslot.reshape(maxtiles, tile, 1)
    grid_spec = pltpu.PrefetchScalarGridSpec(
        num_scalar_prefetch=2,
        grid=(maxtiles,),
        in_specs=[
            pl.BlockSpec((tile, h), lambda i, te, nt: (i, 0)),
            pl.BlockSpec((1, i2, h), lambda i, te, nt: (te[i], 0, 0)),
            pl.BlockSpec((1, h, i_dim), lambda i, te, nt: (te[i], 0, 0)),
            pl.BlockSpec((1, tile, 1), lambda i, te, nt: (i, 0, 0)),
        ],
        out_specs=pl.BlockSpec((tile, h), lambda i, te, nt: (i, 0)),
    )
    return pl.pallas_call(
        _grouped_gemm_body,
        grid_spec=grid_spec,
        out_shape=jax.ShapeDtypeStruct((pad_a, h), jnp.float32),
        compiler_params=pltpu.CompilerParams(
            dimension_semantics=("arbitrary",)),
    )(tile_expert, nt_valid, xg, w13_bf, w2_bf, ws3)


def _shared_mlp_body(x_ref, wgu_ref, wd_ref, out_ref):
    gate_up = jax.lax.dot_general(
        x_ref[...], wgu_ref[...],
        dimension_numbers=(((1,), (1,)), ((), ())),
        preferred_element_type=jnp.float32)
    gate, up = jnp.split(gate_up, 2, axis=-1)
    act = (jax.nn.silu(gate) * up).astype(x_ref.dtype)
    out_ref[...] = jax.lax.dot_general(
        act, wd_ref[...],
        dimension_numbers=(((1,), (1,)), ((), ())),
        preferred_element_type=jnp.float32)


def _shared_mlp(x_bf, sgu_bf, sd_bf, tile_t):
    t, h = x_bf.shape
    i2 = sgu_bf.shape[0]
    i_dim = i2 // 2
    nt = t // tile_t
    return pl.pallas_call(
        _shared_mlp_body,
        grid=(nt,),
        in_specs=[
            pl.BlockSpec((tile_t, h), lambda i: (i, 0)),
            pl.BlockSpec((i2, h), lambda i: (0, 0)),
            pl.BlockSpec((h, i_dim), lambda i: (0, 0)),
        ],
        out_specs=pl.BlockSpec((tile_t, h), lambda i: (i, 0)),
        out_shape=jax.ShapeDtypeStruct((t, h), jnp.float32),
    )(x_bf, sgu_bf, sd_bf)


def _routing_metadata(topk_idx, topk_weights, e, tile, maxtiles, pad_a):
    t, k = topk_idx.shape
    a = t * k
    ti = topk_idx.reshape(-1).astype(jnp.int32)
    onehot = (ti[:, None] == jnp.arange(e, dtype=jnp.int32)[None, :])
    onehot = onehot.astype(jnp.int32)
    counts = onehot.sum(axis=0)
    ranks_excl = jnp.cumsum(onehot, axis=0) - onehot
    rank = jnp.take_along_axis(ranks_excl, ti[:, None], axis=1)[:, 0]
    padded_counts = ((counts + tile - 1) // tile) * tile
    pad_off = jnp.concatenate(
        [jnp.zeros((1,), jnp.int32),
         jnp.cumsum(padded_counts)[:-1].astype(jnp.int32)])
    slot = pad_off[ti] + rank
    gtok = jnp.zeros((pad_a,), jnp.int32).at[slot].set(
        jnp.arange(a, dtype=jnp.int32) // k)
    wslot = jnp.zeros((pad_a,), jnp.float32).at[slot].set(
        topk_weights.reshape(-1).astype(jnp.float32))
    pos = slot.reshape(t, k)
    ntile_e = (padded_counts // tile).astype(jnp.int32)
    cum_tiles = jnp.cumsum(ntile_e)
    tile_expert = jnp.searchsorted(
        cum_tiles, jnp.arange(maxtiles, dtype=jnp.int32), side="right")
    tile_expert = jnp.minimum(tile_expert, e - 1).astype(jnp.int32)
    nt_valid = cum_tiles[-1:].astype(jnp.int32)
    return gtok, wslot, pos, tile_expert, nt_valid


def kernel(hidden_states, topk_idx, topk_weights, w13, w2,
           shared_gate_up, shared_down):
    t, h = hidden_states.shape
    e, i2, _ = w13.shape
    k = topk_idx.shape[1]
    a = t * k
    tile = 256
    maxtiles = a // tile + e
    pad_a = maxtiles * tile

    w13_bf = w13.astype(jnp.bfloat16)
    w2_bf = w2.astype(jnp.bfloat16)
    sgu_bf = shared_gate_up.astype(jnp.bfloat16)
    sd_bf = shared_down.astype(jnp.bfloat16)
    x_bf = hidden_states.astype(jnp.bfloat16)

    gtok, wslot, pos, tile_expert, nt_valid = _routing_metadata(
        topk_idx, topk_weights, e, tile, maxtiles, pad_a)

    xg = _dispatch_sc(hidden_states, gtok, pad_a)
    dg = _grouped_gemm(xg, w13_bf, w2_bf, wslot, tile_expert, nt_valid,
                       tile, maxtiles)
    combined_x = _combine_sc(dg, pos[:, 0], pos[:, 1])
    shared_output = _shared_mlp(x_bf, sgu_bf, sd_bf, min(256, t))

    return (combined_x, shared_output)
